# speedup vs baseline: 2.5479x; 2.5479x over previous
import jax
import jax.numpy as jnp
from jax.experimental import pallas as pl
from jax.experimental.pallas import tpu as pltpu

H = 1024
E = 16
I = 512
N_GROUP = 4
GSIZE = E // N_GROUP
T = 2048
TB = 1024
E_STEPS = E + 2


def _routing_weights(x_f32, gate_w):
    logits = jax.lax.dot_general(
        x_f32, gate_w, (((1,), (1,)), ((), ())),
        preferred_element_type=jnp.float32)
    m = jnp.max(logits, axis=-1, keepdims=True)
    p = jnp.exp(logits - m)
    s = p / jnp.sum(p, axis=-1, keepdims=True)

    lane = jax.lax.broadcasted_iota(jnp.int32, (TB, E), 1)
    grp = lane // GSIZE
    gmax = jnp.zeros_like(s)
    for g in range(N_GROUP):
        mg = jnp.max(jnp.where(grp == g, s, -1.0), axis=-1, keepdims=True)
        gmax = jnp.where(grp == g, mg, gmax)
    vg1 = jnp.max(gmax, axis=-1, keepdims=True)
    l1 = jnp.min(jnp.where(gmax == vg1, lane, E), axis=-1, keepdims=True)
    g1 = l1 // GSIZE
    gmax2 = jnp.where(grp == g1, -1.0, gmax)
    vg2 = jnp.max(gmax2, axis=-1, keepdims=True)
    l2 = jnp.min(jnp.where(gmax2 == vg2, lane, E), axis=-1, keepdims=True)
    g2 = l2 // GSIZE
    ms = jnp.where((grp == g1) | (grp == g2), s, 0.0)
    v1 = jnp.max(ms, axis=-1, keepdims=True)
    i1 = jnp.min(jnp.where(ms == v1, lane, E), axis=-1, keepdims=True)
    ms2 = jnp.where(lane == i1, -1.0, ms)
    v2 = jnp.max(ms2, axis=-1, keepdims=True)
    i2 = jnp.min(jnp.where(ms2 == v2, lane, E), axis=-1, keepdims=True)
    sel = (lane == i1) | (lane == i2)
    return jnp.where(sel, ms, 0.0) / (v1 + v2 + 1e-20)


def _mlp(xb, g_ref, u_ref, d_ref):
    wg = g_ref[0].astype(jnp.bfloat16)
    wu = u_ref[0].astype(jnp.bfloat16)
    wd = d_ref[0].astype(jnp.bfloat16)
    hg = jax.lax.dot_general(xb, wg, (((1,), (1,)), ((), ())),
                             preferred_element_type=jnp.float32)
    hu = jax.lax.dot_general(xb, wu, (((1,), (1,)), ((), ())),
                             preferred_element_type=jnp.float32)
    h = (jax.nn.silu(hg) * hu).astype(jnp.bfloat16)
    return jax.lax.dot_general(h, wd, (((1,), (1,)), ((), ())),
                               preferred_element_type=jnp.float32)


def _moe_kernel(xf_ref, xb_ref, gatew_ref, wg_ref, wu_ref, wd_ref,
                wsg_ref, wsu_ref, wsd_ref, out_ref, w_scr):
    e = pl.program_id(1)

    @pl.when(e == 0)
    def _():
        w_scr[...] = _routing_weights(xf_ref[...], gatew_ref[...])
        out_ref[...] = jnp.zeros_like(out_ref)

    @pl.when(e < E)
    def _():
        lane = jax.lax.broadcasted_iota(jnp.int32, (TB, E), 1)
        w_col = jnp.sum(jnp.where(lane == e, w_scr[...], 0.0),
                        axis=-1, keepdims=True)
        out_ref[...] += w_col * _mlp(xb_ref[...], wg_ref, wu_ref, wd_ref)

    @pl.when(e >= E)
    def _():
        out_ref[...] += _mlp(xb_ref[...], wsg_ref, wsu_ref, wsd_ref)


def kernel(hidden_states, gate_w, w_gate, w_up, w_down,
           ws_gate, ws_up, ws_down):
    xb = hidden_states.astype(jnp.bfloat16)
    wsg = ws_gate.reshape(2, I, H)
    wsu = ws_up.reshape(2, I, H)
    wsd = jnp.stack([ws_down[:, :I], ws_down[:, I:]])

    out = pl.pallas_call(
        _moe_kernel,
        grid=(T // TB, E_STEPS),
        in_specs=[
            pl.BlockSpec((TB, H), lambda h, e: (h, 0)),
            pl.BlockSpec((TB, H), lambda h, e: (h, 0)),
            pl.BlockSpec((E, H), lambda h, e: (0, 0)),
            pl.BlockSpec((1, I, H), lambda h, e: (jnp.minimum(e, E - 1), 0, 0)),
            pl.BlockSpec((1, I, H), lambda h, e: (jnp.minimum(e, E - 1), 0, 0)),
            pl.BlockSpec((1, H, I), lambda h, e: (jnp.minimum(e, E - 1), 0, 0)),
            pl.BlockSpec((1, I, H),
                         lambda h, e: (jnp.clip(e - E, 0, 1), 0, 0)),
            pl.BlockSpec((1, I, H),
                         lambda h, e: (jnp.clip(e - E, 0, 1), 0, 0)),
            pl.BlockSpec((1, H, I),
                         lambda h, e: (jnp.clip(e - E, 0, 1), 0, 0)),
        ],
        out_specs=pl.BlockSpec((TB, H), lambda h, e: (h, 0)),
        out_shape=jax.ShapeDtypeStruct((T, H), jnp.float32),
        scratch_shapes=[pltpu.VMEM((TB, E), jnp.float32)],
        compiler_params=pltpu.CompilerParams(
            dimension_semantics=("parallel", "arbitrary")),
    )(hidden_states, xb, gate_w, w_gate, w_up, w_down, wsg, wsu, wsd)
    return out
